# baseline (speedup 1.0000x reference)
import jax
import jax.numpy as jnp
from jax.experimental import pallas as pl

N = 32768
D = 64
FF = 48

BN = 2048


def _moe_block(x_ref, w1_ref, b1_ref, w2_ref, sb2_ref, rb2_ref, out_ref):
    x = x_ref[...]

    h = jnp.dot(x, w1_ref[...], preferred_element_type=jnp.float32) + b1_ref[...]
    a = h[:, : 3 * FF]
    b = h[:, 3 * FF : 6 * FF]
    l0 = h[:, 6 * FF : 6 * FF + 1]
    l1 = h[:, 6 * FF + 1 : 6 * FF + 2]

    act = (a * jax.nn.sigmoid(a)) * b

    m = (l1 > l0).astype(jnp.float32)
    w = jax.nn.sigmoid(jnp.abs(l1 - l0))
    w0 = w * (1.0 - m)
    w1 = w * m
    col = jax.lax.broadcasted_iota(jnp.int32, (1, 3 * FF), 1)
    scale = jnp.where(col < FF, 1.0, jnp.where(col < 2 * FF, w0, w1))
    act = act * scale

    out = jnp.dot(act, w2_ref[...], preferred_element_type=jnp.float32)
    rb2 = rb2_ref[...]
    out_ref[...] = out + sb2_ref[...] + w * (rb2[0:1] * (1.0 - m) + rb2[1:2] * m)


@jax.jit
def kernel(x, sw1, sb1, sw2, sb2, rw1, rb1, rw2, rb2, gw, gb):
    w1cat = jnp.concatenate(
        [sw1[:, :FF], rw1[0][:, :FF], rw1[1][:, :FF],
         sw1[:, FF:], rw1[0][:, FF:], rw1[1][:, FF:], gw], axis=1)
    b1cat = jnp.concatenate(
        [sb1[:FF], rb1[0][:FF], rb1[1][:FF],
         sb1[FF:], rb1[0][FF:], rb1[1][FF:], gb], axis=0)[None, :]
    w2cat = jnp.concatenate([sw2, rw2[0], rw2[1]], axis=0)

    grid = (N // BN,)
    full = lambda *s: pl.BlockSpec(s, lambda i: (0,) * len(s))
    return pl.pallas_call(
        _moe_block,
        grid=grid,
        in_specs=[
            pl.BlockSpec((BN, D), lambda i: (i, 0)),
            full(D, 6 * FF + 2), full(1, 6 * FF + 2),
            full(3 * FF, D), full(D), full(2, D),
        ],
        out_specs=pl.BlockSpec((BN, D), lambda i: (i, 0)),
        out_shape=jax.ShapeDtypeStruct((N, D), jnp.float32),
    )(x, w1cat, b1cat, w2cat, sb2, rb2)


# speedup vs baseline: 1.2496x; 1.2496x over previous
import jax
import jax.numpy as jnp
from jax.experimental import pallas as pl

N = 32768
D = 64
FF = 48

BN = 4096


def _moe_block(x_ref, sw1_ref, sb1_ref, sw2_ref, sb2_ref,
               rw1_ref, rb1_ref, rw2_ref, rb2_ref, gw_ref, gb_ref,
               out_ref):
    x = x_ref[...]

    def expert(w1, b1, w2, b2):
        h = jnp.dot(x, w1, preferred_element_type=jnp.float32) + b1
        a = h[:, :FF]
        b = h[:, FF:]
        act = (a * jax.nn.sigmoid(a)) * b
        return jnp.dot(act, w2, preferred_element_type=jnp.float32) + b2

    shared = expert(sw1_ref[...], sb1_ref[...], sw2_ref[...], sb2_ref[...])
    o0 = expert(rw1_ref[0], rb1_ref[0], rw2_ref[0], rb2_ref[0])
    o1 = expert(rw1_ref[1], rb1_ref[1], rw2_ref[1], rb2_ref[1])

    logits = jnp.dot(x, gw_ref[...], preferred_element_type=jnp.float32) + gb_ref[...]
    l0 = logits[:, 0:1]
    l1 = logits[:, 1:2]
    pick1 = l1 > l0
    w = jax.nn.sigmoid(jnp.abs(l1 - l0))
    routed = jnp.where(pick1, o1, o0) * w
    out_ref[...] = shared + routed


@jax.jit
def kernel(x, sw1, sb1, sw2, sb2, rw1, rb1, rw2, rb2, gw, gb):
    grid = (N // BN,)
    full = lambda *s: pl.BlockSpec(s, lambda i: (0,) * len(s))
    return pl.pallas_call(
        _moe_block,
        grid=grid,
        in_specs=[
            pl.BlockSpec((BN, D), lambda i: (i, 0)),
            full(D, 2 * FF), full(2 * FF), full(FF, D), full(D),
            full(2, D, 2 * FF), full(2, 2 * FF), full(2, FF, D), full(2, D),
            full(D, 2), full(2),
        ],
        out_specs=pl.BlockSpec((BN, D), lambda i: (i, 0)),
        out_shape=jax.ShapeDtypeStruct((N, D), jnp.float32),
    )(x, sw1, sb1, sw2, sb2, rw1, rb1, rw2, rb2, gw, gb)
